# Initial kernel scaffold; baseline (speedup 1.0000x reference)
import jax
import jax.numpy as jnp
from jax.experimental import pallas as pl


def kernel(x_u, x_v, edge_index_uv, edge_index_vu, Wl, bl, Wr):
    raise NotImplementedError("write your pallas kernel here")


# speedup vs baseline: 3.2581x; 3.2581x over previous
import functools

import jax
import jax.numpy as jnp
from jax.experimental import pallas as pl
from jax.experimental.pallas import tpu as pltpu

_BM = 512
_BK = 512


def _sage_body(a_ref, x_ref, cnt_ref, xd_ref, wl_ref, wr_ref, bl_ref,
               o_ref, acc_ref):
    k = pl.program_id(1)

    @pl.when(k == 0)
    def _init():
        acc_ref[...] = jnp.zeros_like(acc_ref)

    acc_ref[...] += jnp.dot(a_ref[...], x_ref[...],
                            preferred_element_type=jnp.float32)

    @pl.when(k == pl.num_programs(1) - 1)
    def _epilogue():
        mean = acc_ref[...] / jnp.maximum(cnt_ref[...], 1.0)
        o_ref[...] = (jnp.dot(mean, wl_ref[...],
                              preferred_element_type=jnp.float32)
                      + jnp.dot(xd_ref[...], wr_ref[...],
                                preferred_element_type=jnp.float32)
                      + bl_ref[...])


@functools.partial(jax.jit, static_argnames=())
def _sage(a, x_src, cnt, x_dst, wl, wr, bias):
    np_, d = x_src.shape
    grid = (np_ // _BM, np_ // _BK)
    return pl.pallas_call(
        _sage_body,
        grid=grid,
        in_specs=[
            pl.BlockSpec((_BM, _BK), lambda m, k: (m, k)),
            pl.BlockSpec((_BK, d), lambda m, k: (k, 0)),
            pl.BlockSpec((_BM, 1), lambda m, k: (m, 0)),
            pl.BlockSpec((_BM, d), lambda m, k: (m, 0)),
            pl.BlockSpec((d, d), lambda m, k: (0, 0)),
            pl.BlockSpec((d, d), lambda m, k: (0, 0)),
            pl.BlockSpec((1, d), lambda m, k: (0, 0)),
        ],
        out_specs=pl.BlockSpec((_BM, d), lambda m, k: (m, 0)),
        out_shape=jax.ShapeDtypeStruct((np_, d), jnp.float32),
        scratch_shapes=[pltpu.VMEM((_BM, d), jnp.float32)],
        compiler_params=pltpu.CompilerParams(
            dimension_semantics=("parallel", "arbitrary")),
    )(a, x_src, cnt, x_dst, wl, wr, bias)


def kernel(x_u, x_v, edge_index_uv, edge_index_vu, Wl, bl, Wr):
    n_u, d = x_u.shape
    n_v = x_v.shape[0]
    np_ = ((max(n_u, n_v) + _BM - 1) // _BM) * _BM

    xu = jnp.zeros((np_, d), jnp.float32).at[:n_u].set(x_u)
    xv = jnp.zeros((np_, d), jnp.float32).at[:n_v].set(x_v)

    a_uv = jnp.zeros((np_, np_), jnp.float32).at[
        edge_index_uv[1], edge_index_uv[0]].add(1.0)
    a_vu = jnp.zeros((np_, np_), jnp.float32).at[
        edge_index_vu[1], edge_index_vu[0]].add(1.0)
    cnt_v = jnp.zeros((np_, 1), jnp.float32).at[edge_index_uv[1], 0].add(1.0)
    cnt_u = jnp.zeros((np_, 1), jnp.float32).at[edge_index_vu[1], 0].add(1.0)

    num_layers = Wl.shape[0] // 2
    for i in range(num_layers):
        new_v = _sage(a_uv, xu, cnt_v, xv, Wl[2 * i], Wr[2 * i],
                      bl[2 * i][None, :])
        new_u = _sage(a_vu, xv, cnt_u, xu, Wl[2 * i + 1], Wr[2 * i + 1],
                      bl[2 * i + 1][None, :])
        xu, xv = new_u, new_v
    return xu[:n_u], xv[:n_v]
